# Initial kernel scaffold; baseline (speedup 1.0000x reference)
import jax
import jax.numpy as jnp
from jax.experimental import pallas as pl


def kernel(positions, velocities, edge_index, atom_types, bessel_freqs, W_elem, W_init, w_l1, w_l2, W_r1, W_r2, W_mix, W_out):
    raise NotImplementedError("write your pallas kernel here")


# speedup vs baseline: 30.6308x; 30.6308x over previous
import jax
import jax.numpy as jnp
from jax import lax
from jax.experimental import pallas as pl
from jax.experimental.pallas import tpu as pltpu
from jax.experimental.pallas import tpu_sc as plsc

N = 10000
E = 160000
C = 32
RC = 5.0
PCUT = 6
AVG_NEI = 16.0

NP = 10240
EP = 160768
EPR = EP // 128
NCORE = 2
NSUB = 16
NW = NCORE * NSUB
ROW = 128
GR = 72
B = 128
NBTOT = E // B
NROWS_T = NP // NSUB


TE1 = E // NW


def _k1_body(pos_hbm, src_hbm, dst_hbm, ev_hbm, pos_v, src_v, dst_v,
             ex_v, ey_v, ez_v):
    c = lax.axis_index("c")
    s = lax.axis_index("s")
    w = c * NSUB + s
    base = w * TE1
    pltpu.sync_copy(pos_hbm, pos_v)
    pltpu.sync_copy(src_hbm.at[pl.ds(base, TE1)], src_v)
    pltpu.sync_copy(dst_hbm.at[pl.ds(base, TE1)], dst_v)

    ngroups = (TE1 + 15) // 16

    def body(g, _):
        off = jnp.minimum(g * 16, TE1 - 16)
        si = src_v[pl.ds(off, 16)] * 3
        di = dst_v[pl.ds(off, 16)] * 3
        sx = plsc.load_gather(pos_v, [si])
        sy = plsc.load_gather(pos_v, [si + 1])
        sz = plsc.load_gather(pos_v, [si + 2])
        dx = plsc.load_gather(pos_v, [di])
        dy = plsc.load_gather(pos_v, [di + 1])
        dz = plsc.load_gather(pos_v, [di + 2])
        ex_v[pl.ds(off, 16)] = dx - sx
        ey_v[pl.ds(off, 16)] = dy - sy
        ez_v[pl.ds(off, 16)] = dz - sz
        return _

    lax.fori_loop(0, ngroups, body, None)
    pltpu.sync_copy(ex_v, ev_hbm.at[pl.ds(0 * EP + base, TE1)])
    pltpu.sync_copy(ey_v, ev_hbm.at[pl.ds(1 * EP + base, TE1)])
    pltpu.sync_copy(ez_v, ev_hbm.at[pl.ds(2 * EP + base, TE1)])


def _k1(positions, src, dst):
    mesh = plsc.VectorSubcoreMesh(core_axis_name="c", subcore_axis_name="s")
    f = pl.kernel(
        _k1_body,
        out_type=jax.ShapeDtypeStruct((3 * EP,), jnp.float32),
        mesh=mesh,
        compiler_params=pltpu.CompilerParams(needs_layout_passes=False),
        scratch_types=[
            pltpu.VMEM((N * 3,), jnp.float32),
            pltpu.VMEM((TE1,), jnp.int32),
            pltpu.VMEM((TE1,), jnp.int32),
            pltpu.VMEM((TE1,), jnp.float32),
            pltpu.VMEM((TE1,), jnp.float32),
            pltpu.VMEM((TE1,), jnp.float32),
        ],
    )
    return f(positions, src, dst)


def _ka_body(ev_ref, fr_ref, w1_ref, w2_ref, g_ref):
    ev = ev_ref[...]
    ex, ey, ez = ev[0], ev[1], ev[2]
    rr = ex * ex + ey * ey + ez * ez
    r = jnp.sqrt(rr)
    rs = jnp.maximum(r, 1e-9)
    rinv = 1.0 / rs
    u = jnp.clip(r * (1.0 / RC), 0.0, 1.0)
    p = float(PCUT)
    u2 = u * u
    u3 = u2 * u
    u6 = u3 * u3
    u7 = u6 * u
    u8 = u7 * u
    cut = (1.0 - ((p + 1.0) * (p + 2.0) / 2.0) * u6
           + p * (p + 2.0) * u7
           - (p * (p + 1.0) / 2.0) * u8)
    scale = jnp.sqrt(2.0 / RC) * rinv * cut
    embs = []
    for j in range(8):
        fj = fr_ref[0, j]
        embs.append(jnp.sin(fj * rs) * scale)
    emb = jnp.stack(embs).reshape(8, 1024)

    s3 = jnp.sqrt(3.0)
    ye = jnp.stack([s3 * ex * rinv, s3 * ey * rinv, s3 * ez * rinv]
                   ).reshape(3, 1024)
    pad = jnp.zeros((5, 1024), jnp.float32)

    for l in range(2):
        w1 = w1_ref[l]
        h1 = lax.dot_general(w1, emb, (((0,), (0,)), ((), ())),
                             precision=lax.Precision.HIGHEST)
        sact = h1 * jax.nn.sigmoid(h1)
        w2 = w2_ref[l]
        h2 = lax.dot_general(w2, sact, (((0,), (0,)), ((), ())),
                             precision=lax.Precision.HIGHEST)
        g_ref[l] = jnp.concatenate([h2, ye, pad], axis=0)


def _ka(evec3, freqs, W_r1, W_r2):
    grid = (EP // 1024,)
    return pl.pallas_call(
        _ka_body,
        grid=grid,
        in_specs=[
            pl.BlockSpec((3, 8, 128), lambda i: (0, i, 0)),
            pl.BlockSpec((1, 8), lambda i: (0, 0)),
            pl.BlockSpec((2, 8, 16), lambda i: (0, 0, 0)),
            pl.BlockSpec((2, 16, 64), lambda i: (0, 0, 0)),
        ],
        out_specs=pl.BlockSpec((2, GR, 1024), lambda i: (0, 0, i)),
        out_shape=jax.ShapeDtypeStruct((2, GR, EP), jnp.float32),
    )(evec3, freqs, W_r1, W_r2)


def _kb_body(v_ref, m_ref, f_ref):
    v = v_ref[...]
    vn = jnp.sqrt(jnp.sum(v * v, axis=1, keepdims=True))
    u = v * (jnp.sqrt(3.0) / jnp.maximum(vn, 1e-9))
    f_ref[...] = jnp.dot(u, m_ref[...], precision=lax.Precision.HIGHEST)


def _kb(velp, m3):
    grid = (NP // 1024,)
    return pl.pallas_call(
        _kb_body,
        grid=grid,
        in_specs=[
            pl.BlockSpec((1024, 3), lambda i: (i, 0)),
            pl.BlockSpec((3, 128), lambda i: (0, 0)),
        ],
        out_specs=pl.BlockSpec((1024, 128), lambda i: (i, 0)),
        out_shape=jax.ShapeDtypeStruct((NP, 128), jnp.float32),
    )(velp, m3)


def _ksc_body(feat_hbm, src_hbm, dst_hbm, g_hbm, agg_hbm,
              sidx_v, didx_v, g_v, rows_v, msg_v, zbuf_v, agg_sh, sem):
    c = lax.axis_index("c")
    t = lax.axis_index("s")
    w = c * NSUB + t

    zv = jnp.zeros((16,), jnp.float32)
    for e16 in range(B):
        for j in range(2):
            msg_v[e16, pl.ds(96 + j * 16, 16)] = zv
    for i in range(16):
        for j in range(8):
            zbuf_v[i, pl.ds(j * 16, 16)] = zv

    def zbody(j, _):
        pltpu.sync_copy(zbuf_v, agg_sh.at[pl.ds(t * NROWS_T + j * 16, 16)])
        return _

    lax.fori_loop(0, NROWS_T // 16, zbody, None)
    plsc.subcore_barrier()

    ib = lax.iota(jnp.int32, 16)
    nb = jnp.where(w < NBTOT - (NBTOT // NW) * NW, NBTOT // NW + 1, NBTOT // NW)

    def batch(k, _):
        base = (w + k * NW) * B
        pltpu.sync_copy(src_hbm.at[pl.ds(base, B)], sidx_v)
        pltpu.sync_copy(dst_hbm.at[pl.ds(base, B)], didx_v)
        pltpu.sync_copy(g_hbm.at[:, pl.ds(base, B)], g_v)
        pltpu.async_copy(feat_hbm.at[sidx_v], rows_v, sem).wait()

        def edge(e, _):
            ev16 = jnp.full((16,), e, jnp.int32)
            g1a = plsc.load_gather(g_v, [ib, ev16])
            g1b = plsc.load_gather(g_v, [ib + 16, ev16])
            g2a = plsc.load_gather(g_v, [ib + 32, ev16])
            g2b = plsc.load_gather(g_v, [ib + 48, ev16])
            yev = plsc.load_gather(g_v, [ib + 64, ev16])
            for m in range(3):
                fa = rows_v[e, pl.ds(m * 32, 16)]
                fb = rows_v[e, pl.ds(m * 32 + 16, 16)]
                msg_v[e, pl.ds(m * 32, 16)] = fa * g1a + yev[m] * g2a
                msg_v[e, pl.ds(m * 32 + 16, 16)] = fb * g1b + yev[m] * g2b
            return _

        lax.fori_loop(0, B, edge, None)
        pltpu.sync_copy(msg_v, agg_sh.at[didx_v], add=True)
        return _

    lax.fori_loop(0, nb, batch, None)
    plsc.subcore_barrier()
    pltpu.sync_copy(agg_sh.at[pl.ds(t * NROWS_T, NROWS_T)],
                    agg_hbm.at[pl.ds(c * NP + t * NROWS_T, NROWS_T)])


def _ksc(feat, src, dst, gates):
    mesh = plsc.VectorSubcoreMesh(core_axis_name="c", subcore_axis_name="s")
    f = pl.kernel(
        _ksc_body,
        out_type=jax.ShapeDtypeStruct((2 * NP, ROW), jnp.float32),
        mesh=mesh,
        compiler_params=pltpu.CompilerParams(needs_layout_passes=False),
        scratch_types=[
            pltpu.VMEM((B,), jnp.int32),
            pltpu.VMEM((B,), jnp.int32),
            pltpu.VMEM((GR, B), jnp.float32),
            pltpu.VMEM((B, ROW), jnp.float32),
            pltpu.VMEM((B, ROW), jnp.float32),
            pltpu.VMEM((16, ROW), jnp.float32),
            pltpu.VMEM_SHARED((NP, ROW), jnp.float32),
            pltpu.SemaphoreType.DMA,
        ],
    )
    return f(feat, src, dst, gates)


def _kc_body(f_ref, a_ref, w_ref, o_ref):
    a = a_ref[...]
    f = f_ref[...] + (a[0] + a[1]) * (1.0 / AVG_NEI)
    w = w_ref[...]
    ys = [jnp.dot(f[:, m * 32:(m + 1) * 32], w,
                  precision=lax.Precision.HIGHEST) for m in range(3)]
    ys.append(jnp.zeros((1024, 32), jnp.float32))
    o_ref[...] = jnp.concatenate(ys, axis=1)


def _kc(feat, agg, wmix):
    grid = (NP // 1024,)
    return pl.pallas_call(
        _kc_body,
        grid=grid,
        in_specs=[
            pl.BlockSpec((1024, 128), lambda i: (i, 0)),
            pl.BlockSpec((2, 1024, 128), lambda i: (0, i, 0)),
            pl.BlockSpec((32, 32), lambda i: (0, 0)),
        ],
        out_specs=pl.BlockSpec((1024, 128), lambda i: (i, 0)),
        out_shape=jax.ShapeDtypeStruct((NP, 128), jnp.float32),
    )(feat, agg, wmix)


def _kd_body(f_ref, a_ref, w_ref, wo_ref, o_ref):
    a = a_ref[...]
    f = f_ref[...] + (a[0] + a[1]) * (1.0 / AVG_NEI)
    w = w_ref[...]
    wo = wo_ref[...]
    os = [jnp.dot(jnp.dot(f[:, m * 32:(m + 1) * 32], w,
                          precision=lax.Precision.HIGHEST),
                  wo, precision=lax.Precision.HIGHEST) for m in range(3)]
    cols = [os[0][:, 0:1], os[1][:, 0:1], os[2][:, 0:1],
            os[0][:, 1:2], os[1][:, 1:2], os[2][:, 1:2],
            jnp.zeros((1024, 2), jnp.float32)]
    o_ref[...] = jnp.concatenate(cols, axis=1)


def _kd(feat, agg, wmix, wout):
    grid = (NP // 1024,)
    return pl.pallas_call(
        _kd_body,
        grid=grid,
        in_specs=[
            pl.BlockSpec((1024, 128), lambda i: (i, 0)),
            pl.BlockSpec((2, 1024, 128), lambda i: (0, i, 0)),
            pl.BlockSpec((32, 32), lambda i: (0, 0)),
            pl.BlockSpec((32, 2), lambda i: (0, 0)),
        ],
        out_specs=pl.BlockSpec((1024, 8), lambda i: (i, 0)),
        out_shape=jax.ShapeDtypeStruct((NP, 8), jnp.float32),
    )(feat, agg, wmix, wout)


def kernel(positions, velocities, edge_index, atom_types, bessel_freqs,
           W_elem, W_init, w_l1, w_l2, W_r1, W_r2, W_mix, W_out):
    src = edge_index[0].astype(jnp.int32)
    dst = edge_index[1].astype(jnp.int32)

    evec_flat = _k1(positions.reshape(-1), src, dst)
    evec3 = evec_flat.reshape(3, EPR, 128)

    freqs = bessel_freqs.reshape(1, 8)
    gates = _ka(evec3, freqs, W_r1, W_r2)

    velp = jnp.pad(velocities, ((0, NP - N), (0, 0)))
    m3 = jnp.zeros((3, 128), jnp.float32)
    for m in range(3):
        m3 = m3.at[m, m * 32:(m + 1) * 32].set(w_l1)
    feat0 = _kb(velp, m3)

    agg1 = _ksc(feat0, src, dst, gates[0])
    feat1 = _kc(feat0, agg1.reshape(2, NP, ROW), W_mix[0, 1])
    agg2 = _ksc(feat1, src, dst, gates[1])
    out = _kd(feat1, agg2.reshape(2, NP, ROW), W_mix[1, 1], W_out)
    return out[:N, :6]


# speedup vs baseline: 35.8183x; 1.1694x over previous
import jax
import jax.numpy as jnp
from jax import lax
from jax.experimental import pallas as pl
from jax.experimental.pallas import tpu as pltpu
from jax.experimental.pallas import tpu_sc as plsc

N = 10000
E = 160000
C = 32
RC = 5.0
PCUT = 6
AVG_NEI = 16.0

NP = 10240
EP = 160768
EPR = EP // 128
NCORE = 2
NSUB = 16
NW = NCORE * NSUB
ROW = 128
GR = 72
B = 128
NBTOT = E // B
NROWS_T = NP // NSUB


TE1 = E // NW


def _k1_body(pos_hbm, src_hbm, dst_hbm, ev_hbm, pos_v, src_v, dst_v,
             ex_v, ey_v, ez_v):
    c = lax.axis_index("c")
    s = lax.axis_index("s")
    w = c * NSUB + s
    base = w * TE1
    pltpu.sync_copy(pos_hbm, pos_v)
    pltpu.sync_copy(src_hbm.at[pl.ds(base, TE1)], src_v)
    pltpu.sync_copy(dst_hbm.at[pl.ds(base, TE1)], dst_v)

    ngroups = (TE1 + 15) // 16

    def body(g, _):
        off = jnp.minimum(g * 16, TE1 - 16)
        si = src_v[pl.ds(off, 16)] * 3
        di = dst_v[pl.ds(off, 16)] * 3
        sx = plsc.load_gather(pos_v, [si])
        sy = plsc.load_gather(pos_v, [si + 1])
        sz = plsc.load_gather(pos_v, [si + 2])
        dx = plsc.load_gather(pos_v, [di])
        dy = plsc.load_gather(pos_v, [di + 1])
        dz = plsc.load_gather(pos_v, [di + 2])
        ex_v[pl.ds(off, 16)] = dx - sx
        ey_v[pl.ds(off, 16)] = dy - sy
        ez_v[pl.ds(off, 16)] = dz - sz
        return _

    lax.fori_loop(0, ngroups, body, None)
    pltpu.sync_copy(ex_v, ev_hbm.at[pl.ds(0 * EP + base, TE1)])
    pltpu.sync_copy(ey_v, ev_hbm.at[pl.ds(1 * EP + base, TE1)])
    pltpu.sync_copy(ez_v, ev_hbm.at[pl.ds(2 * EP + base, TE1)])


def _k1(positions, src, dst):
    mesh = plsc.VectorSubcoreMesh(core_axis_name="c", subcore_axis_name="s")
    f = pl.kernel(
        _k1_body,
        out_type=jax.ShapeDtypeStruct((3 * EP,), jnp.float32),
        mesh=mesh,
        compiler_params=pltpu.CompilerParams(needs_layout_passes=False),
        scratch_types=[
            pltpu.VMEM((N * 3,), jnp.float32),
            pltpu.VMEM((TE1,), jnp.int32),
            pltpu.VMEM((TE1,), jnp.int32),
            pltpu.VMEM((TE1,), jnp.float32),
            pltpu.VMEM((TE1,), jnp.float32),
            pltpu.VMEM((TE1,), jnp.float32),
        ],
    )
    return f(positions, src, dst)


def _ka_body(ev_ref, fr_ref, w1_ref, w2_ref, g_ref):
    ev = ev_ref[...]
    ex, ey, ez = ev[0], ev[1], ev[2]
    rr = ex * ex + ey * ey + ez * ez
    r = jnp.sqrt(rr)
    rs = jnp.maximum(r, 1e-9)
    rinv = 1.0 / rs
    u = jnp.clip(r * (1.0 / RC), 0.0, 1.0)
    p = float(PCUT)
    u2 = u * u
    u3 = u2 * u
    u6 = u3 * u3
    u7 = u6 * u
    u8 = u7 * u
    cut = (1.0 - ((p + 1.0) * (p + 2.0) / 2.0) * u6
           + p * (p + 2.0) * u7
           - (p * (p + 1.0) / 2.0) * u8)
    scale = jnp.sqrt(2.0 / RC) * rinv * cut
    embs = []
    for j in range(8):
        fj = fr_ref[0, j]
        embs.append(jnp.sin(fj * rs) * scale)
    emb = jnp.stack(embs).reshape(8, 1024)

    s3 = jnp.sqrt(3.0)
    ye = jnp.stack([s3 * ex * rinv, s3 * ey * rinv, s3 * ez * rinv]
                   ).reshape(3, 1024)
    pad = jnp.zeros((5, 1024), jnp.float32)

    for l in range(2):
        w1 = w1_ref[l]
        h1 = lax.dot_general(w1, emb, (((0,), (0,)), ((), ())),
                             precision=lax.Precision.HIGHEST)
        sact = h1 * jax.nn.sigmoid(h1)
        w2 = w2_ref[l]
        h2 = lax.dot_general(w2, sact, (((0,), (0,)), ((), ())),
                             precision=lax.Precision.HIGHEST)
        g_ref[l] = jnp.concatenate([h2, ye, pad], axis=0)


def _ka(evec3, freqs, W_r1, W_r2):
    grid = (EP // 1024,)
    return pl.pallas_call(
        _ka_body,
        grid=grid,
        in_specs=[
            pl.BlockSpec((3, 8, 128), lambda i: (0, i, 0)),
            pl.BlockSpec((1, 8), lambda i: (0, 0)),
            pl.BlockSpec((2, 8, 16), lambda i: (0, 0, 0)),
            pl.BlockSpec((2, 16, 64), lambda i: (0, 0, 0)),
        ],
        out_specs=pl.BlockSpec((2, GR, 1024), lambda i: (0, 0, i)),
        out_shape=jax.ShapeDtypeStruct((2, GR, EP), jnp.float32),
    )(evec3, freqs, W_r1, W_r2)


def _kb_body(v_ref, m_ref, f_ref):
    v = v_ref[...]
    vn = jnp.sqrt(jnp.sum(v * v, axis=1, keepdims=True))
    u = v * (jnp.sqrt(3.0) / jnp.maximum(vn, 1e-9))
    f_ref[...] = jnp.dot(u, m_ref[...], precision=lax.Precision.HIGHEST)


def _kb(velp, m3):
    grid = (NP // 1024,)
    return pl.pallas_call(
        _kb_body,
        grid=grid,
        in_specs=[
            pl.BlockSpec((1024, 3), lambda i: (i, 0)),
            pl.BlockSpec((3, 128), lambda i: (0, 0)),
        ],
        out_specs=pl.BlockSpec((1024, 128), lambda i: (i, 0)),
        out_shape=jax.ShapeDtypeStruct((NP, 128), jnp.float32),
    )(velp, m3)


NBM = NBTOT // NW
NTAIL = NBTOT - NBM * NW


def _ksc_body(feat_hbm, src_hbm, dst_hbm, g_hbm, agg_hbm,
              sidx0_v, sidx1_v, didx_v, g_v, rows0_v, rows1_v,
              zbuf_v, agg_sh, rsem0, rsem1):
    c = lax.axis_index("c")
    t = lax.axis_index("s")
    w = c * NSUB + t

    zv = jnp.zeros((16,), jnp.float32)
    for i in range(16):
        for j in range(8):
            zbuf_v[i, pl.ds(j * 16, 16)] = zv

    def zbody(j, _):
        pltpu.sync_copy(zbuf_v, agg_sh.at[pl.ds(t * NROWS_T + j * 16, 16)])
        return _

    lax.fori_loop(0, NROWS_T // 16, zbody, None)
    plsc.subcore_barrier()

    ib = lax.iota(jnp.int32, 16)
    sx = (sidx0_v, sidx1_v)
    rw = (rows0_v, rows1_v)
    sm = (rsem0, rsem1)

    def ebase(k):
        return (w * NBM + k) * B

    def prefetch(k, p):
        pltpu.sync_copy(src_hbm.at[pl.ds(ebase(k), B)], sx[p])
        pltpu.async_copy(feat_hbm.at[sx[p]], rw[p], sm[p])

    def wait_gather(p):
        pltpu.make_async_copy(feat_hbm.at[sx[p]], rw[p], sm[p]).wait()

    def compute(p):
        rows_v = rw[p]

        def edge(e, _):
            ev16 = jnp.full((16,), e, jnp.int32)
            g1a = plsc.load_gather(g_v, [ib, ev16])
            g1b = plsc.load_gather(g_v, [ib + 16, ev16])
            g2a = plsc.load_gather(g_v, [ib + 32, ev16])
            g2b = plsc.load_gather(g_v, [ib + 48, ev16])
            yev = plsc.load_gather(g_v, [ib + 64, ev16])
            for m in range(3):
                fa = rows_v[e, pl.ds(m * 32, 16)]
                fb = rows_v[e, pl.ds(m * 32 + 16, 16)]
                rows_v[e, pl.ds(m * 32, 16)] = fa * g1a + yev[m] * g2a
                rows_v[e, pl.ds(m * 32 + 16, 16)] = fb * g1b + yev[m] * g2b
            return _

        lax.fori_loop(0, B, edge, None)

    def body(k, p):
        k = jnp.int32(k)
        q = 1 - p

        @pl.when(k + 1 < NBM)
        def _pre():
            prefetch(k + 1, q)

        pltpu.sync_copy(dst_hbm.at[pl.ds(ebase(k), B)], didx_v)
        pltpu.sync_copy(g_hbm.at[:, pl.ds(ebase(k), B)], g_v)
        wait_gather(p)
        compute(p)
        pltpu.sync_copy(rw[p], agg_sh.at[didx_v], add=True)

    prefetch(0, 0)

    def pairs(i, _):
        body(2 * i, 0)
        body(2 * i + 1, 1)
        return _

    lax.fori_loop(0, NBM // 2, pairs, None)
    body(NBM - 1, (NBM - 1) % 2)

    @pl.when(w < NTAIL)
    def _tail():
        bt = (NBM * NW + w) * B
        pltpu.sync_copy(src_hbm.at[pl.ds(bt, B)], sidx0_v)
        pltpu.sync_copy(dst_hbm.at[pl.ds(bt, B)], didx_v)
        pltpu.sync_copy(g_hbm.at[:, pl.ds(bt, B)], g_v)
        pltpu.async_copy(feat_hbm.at[sidx0_v], rows0_v, rsem0).wait()
        compute(0)
        pltpu.sync_copy(rows0_v, agg_sh.at[didx_v], add=True)

    plsc.subcore_barrier()
    pltpu.sync_copy(agg_sh.at[pl.ds(t * NROWS_T, NROWS_T)],
                    agg_hbm.at[pl.ds(c * NP + t * NROWS_T, NROWS_T)])


def _ksc(feat, src, dst, gates):
    mesh = plsc.VectorSubcoreMesh(core_axis_name="c", subcore_axis_name="s")
    f = pl.kernel(
        _ksc_body,
        out_type=jax.ShapeDtypeStruct((2 * NP, ROW), jnp.float32),
        mesh=mesh,
        compiler_params=pltpu.CompilerParams(needs_layout_passes=False),
        scratch_types=[
            pltpu.VMEM((B,), jnp.int32),
            pltpu.VMEM((B,), jnp.int32),
            pltpu.VMEM((B,), jnp.int32),
            pltpu.VMEM((GR, B), jnp.float32),
            pltpu.VMEM((B, ROW), jnp.float32),
            pltpu.VMEM((B, ROW), jnp.float32),
            pltpu.VMEM((16, ROW), jnp.float32),
            pltpu.VMEM_SHARED((NP, ROW), jnp.float32),
            pltpu.SemaphoreType.DMA,
            pltpu.SemaphoreType.DMA,
        ],
    )
    return f(feat, src, dst, gates)


def _kc_body(f_ref, a_ref, w_ref, o_ref):
    a = a_ref[...]
    f = f_ref[...] + (a[0] + a[1]) * (1.0 / AVG_NEI)
    w = w_ref[...]
    ys = [jnp.dot(f[:, m * 32:(m + 1) * 32], w,
                  precision=lax.Precision.HIGHEST) for m in range(3)]
    ys.append(jnp.zeros((1024, 32), jnp.float32))
    o_ref[...] = jnp.concatenate(ys, axis=1)


def _kc(feat, agg, wmix):
    grid = (NP // 1024,)
    return pl.pallas_call(
        _kc_body,
        grid=grid,
        in_specs=[
            pl.BlockSpec((1024, 128), lambda i: (i, 0)),
            pl.BlockSpec((2, 1024, 128), lambda i: (0, i, 0)),
            pl.BlockSpec((32, 32), lambda i: (0, 0)),
        ],
        out_specs=pl.BlockSpec((1024, 128), lambda i: (i, 0)),
        out_shape=jax.ShapeDtypeStruct((NP, 128), jnp.float32),
    )(feat, agg, wmix)


def _kd_body(f_ref, a_ref, w_ref, wo_ref, o_ref):
    a = a_ref[...]
    f = f_ref[...] + (a[0] + a[1]) * (1.0 / AVG_NEI)
    w = w_ref[...]
    wo = wo_ref[...]
    os = [jnp.dot(jnp.dot(f[:, m * 32:(m + 1) * 32], w,
                          precision=lax.Precision.HIGHEST),
                  wo, precision=lax.Precision.HIGHEST) for m in range(3)]
    cols = [os[0][:, 0:1], os[1][:, 0:1], os[2][:, 0:1],
            os[0][:, 1:2], os[1][:, 1:2], os[2][:, 1:2],
            jnp.zeros((1024, 2), jnp.float32)]
    o_ref[...] = jnp.concatenate(cols, axis=1)


def _kd(feat, agg, wmix, wout):
    grid = (NP // 1024,)
    return pl.pallas_call(
        _kd_body,
        grid=grid,
        in_specs=[
            pl.BlockSpec((1024, 128), lambda i: (i, 0)),
            pl.BlockSpec((2, 1024, 128), lambda i: (0, i, 0)),
            pl.BlockSpec((32, 32), lambda i: (0, 0)),
            pl.BlockSpec((32, 2), lambda i: (0, 0)),
        ],
        out_specs=pl.BlockSpec((1024, 8), lambda i: (i, 0)),
        out_shape=jax.ShapeDtypeStruct((NP, 8), jnp.float32),
    )(feat, agg, wmix, wout)


def kernel(positions, velocities, edge_index, atom_types, bessel_freqs,
           W_elem, W_init, w_l1, w_l2, W_r1, W_r2, W_mix, W_out):
    src = edge_index[0].astype(jnp.int32)
    dst = edge_index[1].astype(jnp.int32)

    evec_flat = _k1(positions.reshape(-1), src, dst)
    evec3 = evec_flat.reshape(3, EPR, 128)

    freqs = bessel_freqs.reshape(1, 8)
    gates = _ka(evec3, freqs, W_r1, W_r2)

    velp = jnp.pad(velocities, ((0, NP - N), (0, 0)))
    m3 = jnp.zeros((3, 128), jnp.float32)
    for m in range(3):
        m3 = m3.at[m, m * 32:(m + 1) * 32].set(w_l1)
    feat0 = _kb(velp, m3)

    agg1 = _ksc(feat0, src, dst, gates[0])
    feat1 = _kc(feat0, agg1.reshape(2, NP, ROW), W_mix[0, 1])
    agg2 = _ksc(feat1, src, dst, gates[1])
    out = _kd(feat1, agg2.reshape(2, NP, ROW), W_mix[1, 1], W_out)
    return out[:N, :6]


# speedup vs baseline: 37.6636x; 1.0515x over previous
import jax
import jax.numpy as jnp
from jax import lax
from jax.experimental import pallas as pl
from jax.experimental.pallas import tpu as pltpu
from jax.experimental.pallas import tpu_sc as plsc

N = 10000
E = 160000
C = 32
RC = 5.0
PCUT = 6
AVG_NEI = 16.0

NP = 10240
EP = 160768
EPR = EP // 128
NCORE = 2
NSUB = 16
NW = NCORE * NSUB
ROW = 128
GR = 72
B = 128
NBTOT = E // B
NROWS_T = NP // NSUB


TE1 = E // NW


def _k1_body(pos_hbm, src_hbm, dst_hbm, ev_hbm, pos_v, src_v, dst_v,
             ex_v, ey_v, ez_v):
    c = lax.axis_index("c")
    s = lax.axis_index("s")
    w = c * NSUB + s
    base = w * TE1
    pltpu.sync_copy(pos_hbm, pos_v)
    pltpu.sync_copy(src_hbm.at[pl.ds(base, TE1)], src_v)
    pltpu.sync_copy(dst_hbm.at[pl.ds(base, TE1)], dst_v)

    ngroups = (TE1 + 15) // 16

    def body(g, _):
        off = jnp.minimum(g * 16, TE1 - 16)
        si = src_v[pl.ds(off, 16)] * 3
        di = dst_v[pl.ds(off, 16)] * 3
        sx = plsc.load_gather(pos_v, [si])
        sy = plsc.load_gather(pos_v, [si + 1])
        sz = plsc.load_gather(pos_v, [si + 2])
        dx = plsc.load_gather(pos_v, [di])
        dy = plsc.load_gather(pos_v, [di + 1])
        dz = plsc.load_gather(pos_v, [di + 2])
        ex_v[pl.ds(off, 16)] = dx - sx
        ey_v[pl.ds(off, 16)] = dy - sy
        ez_v[pl.ds(off, 16)] = dz - sz
        return _

    lax.fori_loop(0, ngroups, body, None)
    pltpu.sync_copy(ex_v, ev_hbm.at[pl.ds(0 * EP + base, TE1)])
    pltpu.sync_copy(ey_v, ev_hbm.at[pl.ds(1 * EP + base, TE1)])
    pltpu.sync_copy(ez_v, ev_hbm.at[pl.ds(2 * EP + base, TE1)])


def _k1(positions, src, dst):
    mesh = plsc.VectorSubcoreMesh(core_axis_name="c", subcore_axis_name="s")
    f = pl.kernel(
        _k1_body,
        out_type=jax.ShapeDtypeStruct((3 * EP,), jnp.float32),
        mesh=mesh,
        compiler_params=pltpu.CompilerParams(needs_layout_passes=False),
        scratch_types=[
            pltpu.VMEM((N * 3,), jnp.float32),
            pltpu.VMEM((TE1,), jnp.int32),
            pltpu.VMEM((TE1,), jnp.int32),
            pltpu.VMEM((TE1,), jnp.float32),
            pltpu.VMEM((TE1,), jnp.float32),
            pltpu.VMEM((TE1,), jnp.float32),
        ],
    )
    return f(positions, src, dst)


def _ka_body(ev_ref, fr_ref, w1_ref, w2_ref, g_ref):
    ev = ev_ref[...]
    ex, ey, ez = ev[0], ev[1], ev[2]
    rr = ex * ex + ey * ey + ez * ez
    r = jnp.sqrt(rr)
    rs = jnp.maximum(r, 1e-9)
    rinv = 1.0 / rs
    u = jnp.clip(r * (1.0 / RC), 0.0, 1.0)
    p = float(PCUT)
    u2 = u * u
    u3 = u2 * u
    u6 = u3 * u3
    u7 = u6 * u
    u8 = u7 * u
    cut = (1.0 - ((p + 1.0) * (p + 2.0) / 2.0) * u6
           + p * (p + 2.0) * u7
           - (p * (p + 1.0) / 2.0) * u8)
    scale = jnp.sqrt(2.0 / RC) * rinv * cut
    embs = []
    for j in range(8):
        fj = fr_ref[0, j]
        embs.append(jnp.sin(fj * rs) * scale)
    emb = jnp.stack(embs).reshape(8, 1024)

    s3 = jnp.sqrt(3.0)
    ye = jnp.stack([s3 * ex * rinv, s3 * ey * rinv, s3 * ez * rinv]
                   ).reshape(3, 1024)
    pad = jnp.zeros((5, 1024), jnp.float32)

    for l in range(2):
        w1 = w1_ref[l]
        h1 = lax.dot_general(w1, emb, (((0,), (0,)), ((), ())),
                             precision=lax.Precision.HIGHEST)
        sact = h1 * jax.nn.sigmoid(h1)
        w2 = w2_ref[l]
        h2 = lax.dot_general(w2, sact, (((0,), (0,)), ((), ())),
                             precision=lax.Precision.HIGHEST)
        g_ref[l] = jnp.concatenate([h2, ye, pad], axis=0)


def _ka(evec3, freqs, W_r1, W_r2):
    grid = (EP // 1024,)
    return pl.pallas_call(
        _ka_body,
        grid=grid,
        in_specs=[
            pl.BlockSpec((3, 8, 128), lambda i: (0, i, 0)),
            pl.BlockSpec((1, 8), lambda i: (0, 0)),
            pl.BlockSpec((2, 8, 16), lambda i: (0, 0, 0)),
            pl.BlockSpec((2, 16, 64), lambda i: (0, 0, 0)),
        ],
        out_specs=pl.BlockSpec((2, GR, 1024), lambda i: (0, 0, i)),
        out_shape=jax.ShapeDtypeStruct((2, GR, EP), jnp.float32),
    )(evec3, freqs, W_r1, W_r2)


def _kb_body(v_ref, m_ref, f_ref):
    v = v_ref[...]
    vn = jnp.sqrt(jnp.sum(v * v, axis=1, keepdims=True))
    u = v * (jnp.sqrt(3.0) / jnp.maximum(vn, 1e-9))
    f_ref[...] = jnp.dot(u, m_ref[...], precision=lax.Precision.HIGHEST)


def _kb(velp, m3):
    grid = (NP // 1024,)
    return pl.pallas_call(
        _kb_body,
        grid=grid,
        in_specs=[
            pl.BlockSpec((1024, 3), lambda i: (i, 0)),
            pl.BlockSpec((3, 128), lambda i: (0, 0)),
        ],
        out_specs=pl.BlockSpec((1024, 128), lambda i: (i, 0)),
        out_shape=jax.ShapeDtypeStruct((NP, 128), jnp.float32),
    )(velp, m3)


NBM = NBTOT // NW
NTAIL = NBTOT - NBM * NW


def _ksc_body(feat_hbm, src_hbm, dst_hbm, g_hbm, agg_hbm,
              sidx0_v, sidx1_v, didx0_v, didx1_v, g_v, rows0_v, rows1_v,
              zbuf_v, agg_sh, rsem0, rsem1, ssem0, ssem1):
    c = lax.axis_index("c")
    t = lax.axis_index("s")
    w = c * NSUB + t

    zv = jnp.zeros((16,), jnp.float32)
    for i in range(16):
        for j in range(8):
            zbuf_v[i, pl.ds(j * 16, 16)] = zv

    def zbody(j, _):
        pltpu.sync_copy(zbuf_v, agg_sh.at[pl.ds(t * NROWS_T + j * 16, 16)])
        return _

    lax.fori_loop(0, NROWS_T // 16, zbody, None)
    plsc.subcore_barrier()

    ib = lax.iota(jnp.int32, 16)
    sx = (sidx0_v, sidx1_v)
    dx = (didx0_v, didx1_v)
    rw = (rows0_v, rows1_v)
    rs = (rsem0, rsem1)
    ss = (ssem0, ssem1)

    def ebase(k):
        return (w * NBM + k) * B

    def issue_gather(k, p):
        pltpu.sync_copy(src_hbm.at[pl.ds(ebase(k), B)], sx[p])
        pltpu.async_copy(feat_hbm.at[sx[p]], rw[p], rs[p])

    def wait_gather(p):
        pltpu.make_async_copy(feat_hbm.at[sx[p]], rw[p], rs[p]).wait()

    def issue_scatter(p):
        pltpu.async_copy(rw[p], agg_sh.at[dx[p]], ss[p], add=True)

    def wait_scatter(p):
        pltpu.make_async_copy(rw[p], agg_sh.at[dx[p]], ss[p]).wait()

    def compute(p):
        rows_v = rw[p]

        def edge(e, _):
            ev16 = jnp.full((16,), e, jnp.int32)
            g1a = plsc.load_gather(g_v, [ib, ev16])
            g1b = plsc.load_gather(g_v, [ib + 16, ev16])
            g2a = plsc.load_gather(g_v, [ib + 32, ev16])
            g2b = plsc.load_gather(g_v, [ib + 48, ev16])
            yev = plsc.load_gather(g_v, [ib + 64, ev16])
            for m in range(3):
                fa = rows_v[e, pl.ds(m * 32, 16)]
                fb = rows_v[e, pl.ds(m * 32 + 16, 16)]
                rows_v[e, pl.ds(m * 32, 16)] = fa * g1a + yev[m] * g2a
                rows_v[e, pl.ds(m * 32 + 16, 16)] = fb * g1b + yev[m] * g2b
            return _

        lax.fori_loop(0, B, edge, None)

    def body(k, p):
        k = jnp.int32(k)
        q = 1 - p
        pltpu.sync_copy(dst_hbm.at[pl.ds(ebase(k), B)], dx[p])
        pltpu.sync_copy(g_hbm.at[:, pl.ds(ebase(k), B)], g_v)
        wait_gather(p)

        @pl.when(k >= 1)
        def _drain():
            wait_scatter(q)

        @pl.when(k + 1 < NBM)
        def _pre():
            issue_gather(k + 1, q)

        compute(p)
        issue_scatter(p)

    issue_gather(0, 0)

    def pairs(i, _):
        body(2 * i, 0)
        body(2 * i + 1, 1)
        return _

    lax.fori_loop(0, NBM // 2, pairs, None)
    body(NBM - 1, (NBM - 1) % 2)
    wait_scatter((NBM - 1) % 2)

    @pl.when(w < NTAIL)
    def _tail():
        bt = (NBM * NW + w) * B
        pltpu.sync_copy(src_hbm.at[pl.ds(bt, B)], sidx0_v)
        pltpu.sync_copy(dst_hbm.at[pl.ds(bt, B)], didx0_v)
        pltpu.sync_copy(g_hbm.at[:, pl.ds(bt, B)], g_v)
        pltpu.async_copy(feat_hbm.at[sidx0_v], rows0_v, rsem0).wait()
        compute(0)
        pltpu.sync_copy(rows0_v, agg_sh.at[didx0_v], add=True)

    plsc.subcore_barrier()
    pltpu.sync_copy(agg_sh.at[pl.ds(t * NROWS_T, NROWS_T)],
                    agg_hbm.at[pl.ds(c * NP + t * NROWS_T, NROWS_T)])


def _ksc(feat, src, dst, gates):
    mesh = plsc.VectorSubcoreMesh(core_axis_name="c", subcore_axis_name="s")
    f = pl.kernel(
        _ksc_body,
        out_type=jax.ShapeDtypeStruct((2 * NP, ROW), jnp.float32),
        mesh=mesh,
        compiler_params=pltpu.CompilerParams(needs_layout_passes=False),
        scratch_types=[
            pltpu.VMEM((B,), jnp.int32),
            pltpu.VMEM((B,), jnp.int32),
            pltpu.VMEM((B,), jnp.int32),
            pltpu.VMEM((B,), jnp.int32),
            pltpu.VMEM((GR, B), jnp.float32),
            pltpu.VMEM((B, ROW), jnp.float32),
            pltpu.VMEM((B, ROW), jnp.float32),
            pltpu.VMEM((16, ROW), jnp.float32),
            pltpu.VMEM_SHARED((NP, ROW), jnp.float32),
            pltpu.SemaphoreType.DMA,
            pltpu.SemaphoreType.DMA,
            pltpu.SemaphoreType.DMA,
            pltpu.SemaphoreType.DMA,
        ],
    )
    return f(feat, src, dst, gates)


def _kc_body(f_ref, a_ref, w_ref, o_ref):
    a = a_ref[...]
    f = f_ref[...] + (a[0] + a[1]) * (1.0 / AVG_NEI)
    w = w_ref[...]
    ys = [jnp.dot(f[:, m * 32:(m + 1) * 32], w,
                  precision=lax.Precision.HIGHEST) for m in range(3)]
    ys.append(jnp.zeros((1024, 32), jnp.float32))
    o_ref[...] = jnp.concatenate(ys, axis=1)


def _kc(feat, agg, wmix):
    grid = (NP // 1024,)
    return pl.pallas_call(
        _kc_body,
        grid=grid,
        in_specs=[
            pl.BlockSpec((1024, 128), lambda i: (i, 0)),
            pl.BlockSpec((2, 1024, 128), lambda i: (0, i, 0)),
            pl.BlockSpec((32, 32), lambda i: (0, 0)),
        ],
        out_specs=pl.BlockSpec((1024, 128), lambda i: (i, 0)),
        out_shape=jax.ShapeDtypeStruct((NP, 128), jnp.float32),
    )(feat, agg, wmix)


def _kd_body(f_ref, a_ref, w_ref, wo_ref, o_ref):
    a = a_ref[...]
    f = f_ref[...] + (a[0] + a[1]) * (1.0 / AVG_NEI)
    w = w_ref[...]
    wo = wo_ref[...]
    os = [jnp.dot(jnp.dot(f[:, m * 32:(m + 1) * 32], w,
                          precision=lax.Precision.HIGHEST),
                  wo, precision=lax.Precision.HIGHEST) for m in range(3)]
    cols = [os[0][:, 0:1], os[1][:, 0:1], os[2][:, 0:1],
            os[0][:, 1:2], os[1][:, 1:2], os[2][:, 1:2],
            jnp.zeros((1024, 2), jnp.float32)]
    o_ref[...] = jnp.concatenate(cols, axis=1)


def _kd(feat, agg, wmix, wout):
    grid = (NP // 1024,)
    return pl.pallas_call(
        _kd_body,
        grid=grid,
        in_specs=[
            pl.BlockSpec((1024, 128), lambda i: (i, 0)),
            pl.BlockSpec((2, 1024, 128), lambda i: (0, i, 0)),
            pl.BlockSpec((32, 32), lambda i: (0, 0)),
            pl.BlockSpec((32, 2), lambda i: (0, 0)),
        ],
        out_specs=pl.BlockSpec((1024, 8), lambda i: (i, 0)),
        out_shape=jax.ShapeDtypeStruct((NP, 8), jnp.float32),
    )(feat, agg, wmix, wout)


def kernel(positions, velocities, edge_index, atom_types, bessel_freqs,
           W_elem, W_init, w_l1, w_l2, W_r1, W_r2, W_mix, W_out):
    src = edge_index[0].astype(jnp.int32)
    dst = edge_index[1].astype(jnp.int32)

    evec_flat = _k1(positions.reshape(-1), src, dst)
    evec3 = evec_flat.reshape(3, EPR, 128)

    freqs = bessel_freqs.reshape(1, 8)
    gates = _ka(evec3, freqs, W_r1, W_r2)

    velp = jnp.pad(velocities, ((0, NP - N), (0, 0)))
    m3 = jnp.zeros((3, 128), jnp.float32)
    for m in range(3):
        m3 = m3.at[m, m * 32:(m + 1) * 32].set(w_l1)
    feat0 = _kb(velp, m3)

    agg1 = _ksc(feat0, src, dst, gates[0])
    feat1 = _kc(feat0, agg1.reshape(2, NP, ROW), W_mix[0, 1])
    agg2 = _ksc(feat1, src, dst, gates[1])
    out = _kd(feat1, agg2.reshape(2, NP, ROW), W_mix[1, 1], W_out)
    return out[:N, :6]
